# baseline (speedup 1.0000x reference)
import functools

import jax
import jax.numpy as jnp
from jax import lax
from jax.experimental import pallas as pl
from jax.experimental.pallas import tpu as pltpu
from jax.experimental.pallas import tpu_sc as plsc

_B = 16384
_NC = 2
_NS = 16
_NW = _NC * _NS
_BPW = _B // _NW

_D_USER = 32
_D_CAT = 16
_NUSER = 1000000
_NCAT = 1000
_UGRP = 128 // _D_USER
_CGRP = 128 // _D_CAT

_TLB = 8192
_TQ = _TLB // 4
_TNB = (_NUSER + _TLB - 1) // _TLB


def _tc_transpose(ut, eye32):
    def body(in_r, i_r, out_r):
        x = in_r[...]
        eye = i_r[...]
        base = pl.program_id(0) * _TLB
        rr = lax.broadcasted_iota(jnp.int32, (_TQ, 1), 0)
        acc = None
        for k in range(4):
            y = lax.dot_general(
                x[:, k * _TQ:(k + 1) * _TQ], eye[0:_D_USER, 0:_D_USER],
                (((0,), (0,)), ((), ())),
                preferred_element_type=jnp.float32)
            y = jnp.where(base + k * _TQ + rr < _NUSER, y, 0.0)
            z = lax.dot_general(
                y, eye[k * _D_USER:(k + 1) * _D_USER, :],
                (((1,), (0,)), ((), ())),
                preferred_element_type=jnp.float32)
            acc = z if acc is None else acc + z
        out_r[...] = acc

    return pl.pallas_call(
        body,
        grid=(_TNB,),
        in_specs=[pl.BlockSpec((_D_USER, _TLB), lambda i: (0, i)),
                  pl.BlockSpec((128, 128), lambda i: (0, 0))],
        out_specs=pl.BlockSpec((_TQ, 128), lambda i: (i, 0)),
        out_shape=jax.ShapeDtypeStruct((_TQ * _TNB, 128), jnp.float32),
        compiler_params=pltpu.CompilerParams(
            dimension_semantics=("parallel",)),
    )(ut, eye32)


def _sc_gather(user4, cat8, qrow, arow, crow):
    mesh = plsc.VectorSubcoreMesh(core_axis_name="c", subcore_axis_name="s")
    out_type = (
        jax.ShapeDtypeStruct((_NW, _BPW, 128), jnp.float32),
        jax.ShapeDtypeStruct((_NW, _BPW, 128), jnp.float32),
        jax.ShapeDtypeStruct((_NW, _BPW, 128), jnp.float32),
    )

    @functools.partial(
        pl.kernel,
        out_type=out_type,
        mesh=mesh,
        scratch_types=[
            pltpu.VMEM((_BPW,), jnp.int32),
            pltpu.VMEM((_BPW,), jnp.int32),
            pltpu.VMEM((_BPW,), jnp.int32),
            pltpu.VMEM((_BPW, 128), jnp.float32),
            pltpu.SemaphoreType.DMA,
        ],
        compiler_params=pltpu.CompilerParams(use_tc_tiling_on_sc=True),
    )
    def sc_k(u4_h, c8_h, qi_h, ai_h, ci_h, oq, oa, oc,
             qi_v, ai_v, ci_v, rows_v, sem):
        wid = lax.axis_index("s") * _NC + lax.axis_index("c")
        pltpu.sync_copy(qi_h.at[wid], qi_v)
        pltpu.sync_copy(ai_h.at[wid], ai_v)
        pltpu.sync_copy(ci_h.at[wid], ci_v)
        pltpu.async_copy(u4_h.at[qi_v], rows_v, sem).wait()
        pltpu.sync_copy(rows_v, oq.at[wid])
        pltpu.async_copy(u4_h.at[ai_v], rows_v, sem).wait()
        pltpu.sync_copy(rows_v, oa.at[wid])
        pltpu.async_copy(c8_h.at[ci_v], rows_v, sem).wait()
        pltpu.sync_copy(rows_v, oc.at[wid])

    return sc_k(user4, cat8, qrow, arow, crow)


_BS = _BPW
_NBLK = _B // _BS


def _onehot(idx2, n):
    return (idx2 == lax.broadcasted_iota(jnp.int32, (1, n), 1)).astype(jnp.float32)


def _extract(groups, sub, d):
    out = jnp.zeros((groups.shape[0], d), jnp.float32)
    for k in range(128 // d):
        out = out + jnp.where(sub == k, groups[:, k * d:(k + 1) * d], 0.0)
    return out


def _dot(a, b):
    return lax.dot_general(
        a, b, (((1,), (0,)), ((), ())),
        preferred_element_type=jnp.float32)


def _tc_body(qg_r, ag_r, cg_r, numq_r, idx8_r,
             eg_r, eh_r, ew_r, wq1_r, bq1_r, wq2_r, bq2_r,
             wa1_r, ba1_r, wa2_r, ba2_r, qo_r, ao_r):
    idx8 = idx8_r[0]
    wq1 = wq1_r[...]
    f_gen_q = _dot(eg_r[...], wq1[32:40, :])
    f_hour = _dot(eh_r[...], wq1[56:64, :])
    f_week = _dot(ew_r[...], wq1[64:72, :])
    f_gen_a = _dot(eg_r[...], wa1_r[...][32:40, :])

    quser = _extract(qg_r[0], idx8[:, 4:5], _D_USER)
    auser = _extract(ag_r[0], idx8[:, 5:6], _D_USER)
    cvec = cg_r[0][:, 0:_D_CAT]

    qpre = (_dot(quser, wq1[0:32, :])
            + _dot(_onehot(idx8[:, 2:3], 8), f_gen_q)
            + _dot(cvec, wq1[40:56, :])
            + _dot(_onehot(idx8[:, 0:1], 24), f_hour)
            + _dot(_onehot(idx8[:, 1:2], 7), f_week)
            + _dot(numq_r[...], wq1[72:88, :])
            + bq1_r[...])
    qh = jnp.maximum(qpre, 0.0)
    qo_r[...] = _dot(qh, wq2_r[...]) + bq2_r[...]

    apre = (_dot(auser, wa1_r[...][0:32, :])
            + _dot(_onehot(idx8[:, 3:4], 8), f_gen_a)
            + ba1_r[...])
    ah = jnp.maximum(apre, 0.0)
    ao_r[...] = _dot(ah, wa2_r[...]) + ba2_r[...]


def _tc_mlp(qg, ag, cg, num_q, idx8,
            emb_gen, emb_hour, emb_week,
            Wq1, bq1, Wq2, bq2, Wa1, ba1, Wa2, ba2):
    bspec = lambda d: pl.BlockSpec((_BS, d), lambda i: (i, 0))
    gspec = pl.BlockSpec((1, _BPW, 128), lambda i: (i, 0, 0))
    ispec = pl.BlockSpec((1, _BS, 8), lambda i: (i, 0, 0))
    full = lambda s: pl.BlockSpec(s, lambda i: (0,) * len(s))
    return pl.pallas_call(
        _tc_body,
        grid=(_NBLK,),
        in_specs=[
            gspec, gspec, gspec, bspec(16),
            ispec,
            full((8, 8)), full((24, 8)), full((7, 8)),
            full((88, 128)), full((1, 128)), full((128, 128)), full((1, 128)),
            full((40, 128)), full((1, 128)), full((128, 128)), full((1, 128)),
        ],
        out_specs=[bspec(128), bspec(128)],
        out_shape=[
            jax.ShapeDtypeStruct((_B, 128), jnp.float32),
            jax.ShapeDtypeStruct((_B, 128), jnp.float32),
        ],
        compiler_params=pltpu.CompilerParams(
            dimension_semantics=("parallel",)),
    )(qg, ag, cg, num_q, idx8,
      emb_gen, emb_hour, emb_week,
      Wq1, bq1, Wq2, bq2, Wa1, ba1, Wa2, ba2)


def kernel(cat_q, num_q, question_user, num_qu, answer_user, num_au,
           emb_user, emb_gen, emb_cat, emb_hour, emb_week,
           Wq1, bq1, Wq2, bq2, Wa1, ba1, Wa2, ba2):
    del num_qu, num_au
    cat = cat_q[:, 0]
    hour = cat_q[:, 1]
    week = cat_q[:, 2]
    q_uid = question_user[:, 0]
    q_gen = question_user[:, 1]
    a_uid = answer_user[:, 0]
    a_gen = answer_user[:, 1]

    user4 = _tc_transpose(emb_user.T, jnp.eye(128, dtype=jnp.float32))
    cat8 = jnp.pad(emb_cat, ((0, 0), (0, 128 - _D_CAT)))
    qrow = _TQ * (q_uid >> 13) + (q_uid & (_TQ - 1))
    arow = _TQ * (a_uid >> 13) + (a_uid & (_TQ - 1))
    qg, ag, cg = _sc_gather(
        user4, cat8,
        qrow.reshape(_NW, _BPW),
        arow.reshape(_NW, _BPW),
        cat.reshape(_NW, _BPW))

    idx8 = jnp.stack(
        [hour, week, q_gen, a_gen, (q_uid >> 11) & 3, (a_uid >> 11) & 3,
         hour, week],
        axis=1).reshape(_NBLK, _BS, 8)

    q_out, a_out = _tc_mlp(
        qg, ag, cg, num_q, idx8,
        emb_gen, emb_hour, emb_week,
        Wq1, bq1.reshape(1, 128), Wq2, bq2.reshape(1, 128),
        Wa1, ba1.reshape(1, 128), Wa2, ba2.reshape(1, 128))
    return (q_out, a_out)


# speedup vs baseline: 1.1365x; 1.1365x over previous
import functools

import jax
import jax.numpy as jnp
from jax import lax
from jax.experimental import pallas as pl
from jax.experimental.pallas import tpu as pltpu
from jax.experimental.pallas import tpu_sc as plsc

_B = 16384
_NC = 2
_NS = 16
_NW = _NC * _NS
_BPW = _B // _NW

_D_USER = 32
_D_CAT = 16
_NUSER = 1000000
_NCAT = 1000
_UGRP = 128 // _D_USER
_CGRP = 128 // _D_CAT

_TLB = 8192
_TNB = (_NUSER + _TLB - 1) // _TLB


def _tc_transpose(ut, eye32):
    def body(in_r, i_r, out_r):
        y = lax.dot_general(
            in_r[...], i_r[...], (((0,), (0,)), ((), ())),
            preferred_element_type=jnp.float32)
        out_r[:, 0:_D_USER] = y

    return pl.pallas_call(
        body,
        grid=(_TNB,),
        in_specs=[pl.BlockSpec((_D_USER, _TLB), lambda i: (0, i)),
                  pl.BlockSpec((_D_USER, _D_USER), lambda i: (0, 0))],
        out_specs=pl.BlockSpec((_TLB, 128), lambda i: (i, 0)),
        out_shape=jax.ShapeDtypeStruct((_NUSER, 128), jnp.float32),
        compiler_params=pltpu.CompilerParams(
            dimension_semantics=("parallel",)),
    )(ut, eye32)


def _sc_gather(user4, cat8, qrow, arow, crow):
    mesh = plsc.VectorSubcoreMesh(core_axis_name="c", subcore_axis_name="s")
    out_type = (
        jax.ShapeDtypeStruct((_NW, _BPW, 128), jnp.float32),
        jax.ShapeDtypeStruct((_NW, _BPW, 128), jnp.float32),
        jax.ShapeDtypeStruct((_NW, _BPW, 128), jnp.float32),
    )

    @functools.partial(
        pl.kernel,
        out_type=out_type,
        mesh=mesh,
        scratch_types=[
            pltpu.VMEM((_BPW,), jnp.int32),
            pltpu.VMEM((_BPW,), jnp.int32),
            pltpu.VMEM((_BPW,), jnp.int32),
            pltpu.VMEM((_BPW, 128), jnp.float32),
            pltpu.SemaphoreType.DMA,
        ],
        compiler_params=pltpu.CompilerParams(use_tc_tiling_on_sc=True),
    )
    def sc_k(u4_h, c8_h, qi_h, ai_h, ci_h, oq, oa, oc,
             qi_v, ai_v, ci_v, rows_v, sem):
        wid = lax.axis_index("s") * _NC + lax.axis_index("c")
        pltpu.sync_copy(qi_h.at[wid], qi_v)
        pltpu.sync_copy(ai_h.at[wid], ai_v)
        pltpu.sync_copy(ci_h.at[wid], ci_v)
        pltpu.async_copy(u4_h.at[qi_v], rows_v, sem).wait()
        pltpu.sync_copy(rows_v, oq.at[wid])
        pltpu.async_copy(u4_h.at[ai_v], rows_v, sem).wait()
        pltpu.sync_copy(rows_v, oa.at[wid])
        pltpu.async_copy(c8_h.at[ci_v], rows_v, sem).wait()
        pltpu.sync_copy(rows_v, oc.at[wid])

    return sc_k(user4, cat8, qrow, arow, crow)


_BS = _BPW
_NBLK = _B // _BS


def _onehot(idx2, n):
    return (idx2 == lax.broadcasted_iota(jnp.int32, (1, n), 1)).astype(jnp.float32)


def _extract(groups, sub, d):
    out = jnp.zeros((groups.shape[0], d), jnp.float32)
    for k in range(128 // d):
        m = (sub == k).astype(jnp.float32)
        out = out + m * groups[:, k * d:(k + 1) * d]
    return out


def _dot(a, b):
    return lax.dot_general(
        a, b, (((1,), (0,)), ((), ())),
        preferred_element_type=jnp.float32)


def _tc_body(qg_r, ag_r, cg_r, numq_r, idx8_r,
             eg_r, eh_r, ew_r, wq1_r, bq1_r, wq2_r, bq2_r,
             wa1_r, ba1_r, wa2_r, ba2_r, qo_r, ao_r):
    idx8 = idx8_r[0]
    wq1 = wq1_r[...]
    f_gen_q = _dot(eg_r[...], wq1[32:40, :])
    f_hour = _dot(eh_r[...], wq1[56:64, :])
    f_week = _dot(ew_r[...], wq1[64:72, :])
    f_gen_a = _dot(eg_r[...], wa1_r[...][32:40, :])

    quser = qg_r[0][:, 0:_D_USER]
    auser = ag_r[0][:, 0:_D_USER]
    cvec = cg_r[0][:, 0:_D_CAT]

    qpre = (_dot(quser, wq1[0:32, :])
            + _dot(_onehot(idx8[:, 2:3], 8), f_gen_q)
            + _dot(cvec, wq1[40:56, :])
            + _dot(_onehot(idx8[:, 0:1], 24), f_hour)
            + _dot(_onehot(idx8[:, 1:2], 7), f_week)
            + _dot(numq_r[...], wq1[72:88, :])
            + bq1_r[...])
    qh = jnp.maximum(qpre, 0.0)
    qo_r[...] = _dot(qh, wq2_r[...]) + bq2_r[...]

    apre = (_dot(auser, wa1_r[...][0:32, :])
            + _dot(_onehot(idx8[:, 3:4], 8), f_gen_a)
            + ba1_r[...])
    ah = jnp.maximum(apre, 0.0)
    ao_r[...] = _dot(ah, wa2_r[...]) + ba2_r[...]


def _tc_mlp(qg, ag, cg, num_q, idx8,
            emb_gen, emb_hour, emb_week,
            Wq1, bq1, Wq2, bq2, Wa1, ba1, Wa2, ba2):
    bspec = lambda d: pl.BlockSpec((_BS, d), lambda i: (i, 0))
    gspec = pl.BlockSpec((1, _BPW, 128), lambda i: (i, 0, 0))
    ispec = pl.BlockSpec((1, _BS, 8), lambda i: (i, 0, 0))
    full = lambda s: pl.BlockSpec(s, lambda i: (0,) * len(s))
    return pl.pallas_call(
        _tc_body,
        grid=(_NBLK,),
        in_specs=[
            gspec, gspec, gspec, bspec(16),
            ispec,
            full((8, 8)), full((24, 8)), full((7, 8)),
            full((88, 128)), full((1, 128)), full((128, 128)), full((1, 128)),
            full((40, 128)), full((1, 128)), full((128, 128)), full((1, 128)),
        ],
        out_specs=[bspec(128), bspec(128)],
        out_shape=[
            jax.ShapeDtypeStruct((_B, 128), jnp.float32),
            jax.ShapeDtypeStruct((_B, 128), jnp.float32),
        ],
        compiler_params=pltpu.CompilerParams(
            dimension_semantics=("parallel",)),
    )(qg, ag, cg, num_q, idx8,
      emb_gen, emb_hour, emb_week,
      Wq1, bq1, Wq2, bq2, Wa1, ba1, Wa2, ba2)


def kernel(cat_q, num_q, question_user, num_qu, answer_user, num_au,
           emb_user, emb_gen, emb_cat, emb_hour, emb_week,
           Wq1, bq1, Wq2, bq2, Wa1, ba1, Wa2, ba2):
    del num_qu, num_au
    cat = cat_q[:, 0]
    hour = cat_q[:, 1]
    week = cat_q[:, 2]
    q_uid = question_user[:, 0]
    q_gen = question_user[:, 1]
    a_uid = answer_user[:, 0]
    a_gen = answer_user[:, 1]

    user4 = _tc_transpose(emb_user.T, jnp.eye(_D_USER, dtype=jnp.float32))
    cat8 = jnp.pad(emb_cat, ((0, 0), (0, 128 - _D_CAT)))
    qg, ag, cg = _sc_gather(
        user4, cat8,
        q_uid.reshape(_NW, _BPW),
        a_uid.reshape(_NW, _BPW),
        cat.reshape(_NW, _BPW))

    idx8 = jnp.stack(
        [hour, week, q_gen, a_gen, hour, week, q_gen, a_gen],
        axis=1).reshape(_NBLK, _BS, 8)

    q_out, a_out = _tc_mlp(
        qg, ag, cg, num_q, idx8,
        emb_gen, emb_hour, emb_week,
        Wq1, bq1.reshape(1, 128), Wq2, bq2.reshape(1, 128),
        Wa1, ba1.reshape(1, 128), Wa2, ba2.reshape(1, 128))
    return (q_out, a_out)
